# baseline (speedup 1.0000x reference)
import functools

import jax
import jax.numpy as jnp
from jax import lax
from jax.experimental import pallas as pl
from jax.experimental.pallas import tpu as pltpu
from jax.experimental.pallas import tpu_sc as plsc

D_MODEL = 64
NC, NS = 2, 16
NW = NC * NS
ROW = 128
G = 2
TCOLS = 2048


def _transpose_block(in_ref, out_ref):
  x = in_ref[...]
  y = x.T
  out_ref[...] = jnp.concatenate([y, y], axis=1)


def _transpose_table(table_t):
  V = table_t.shape[1]
  steps = (V + TCOLS - 1) // TCOLS
  return pl.pallas_call(
      _transpose_block,
      grid=(steps,),
      in_specs=[pl.BlockSpec((D_MODEL, TCOLS), lambda i: (0, i))],
      out_specs=pl.BlockSpec((TCOLS, 2 * D_MODEL), lambda i: (i, 0)),
      out_shape=jax.ShapeDtypeStruct((V, 2 * D_MODEL), jnp.float32),
  )(table_t)


def _make_gather(B: int):
  rows_total = B // ROW
  rows_per_w = rows_total // NW
  groups = rows_per_w // G

  mesh = plsc.VectorSubcoreMesh(core_axis_name="c", subcore_axis_name="s")

  @functools.partial(
      pl.kernel,
      mesh=mesh,
      compiler_params=pltpu.CompilerParams(use_tc_tiling_on_sc=False),
      out_type=jax.ShapeDtypeStruct((rows_total, ROW, 2 * D_MODEL),
                                    jnp.float32),
      scratch_types=[
          pltpu.VMEM((2, G, ROW), jnp.int32),
          pltpu.VMEM((2, G, ROW, 2 * D_MODEL), jnp.float32),
          pltpu.SemaphoreType.DMA,
          pltpu.SemaphoreType.DMA,
      ],
  )
  def k(idx_hbm, table_hbm, out_hbm, idx_v, rows_v, gsem0, gsem1):
    wid = lax.axis_index("s") * NC + lax.axis_index("c")
    base_w = wid * rows_per_w
    sems = (gsem0, gsem1)

    def load_fire(b, g):
      pltpu.sync_copy(idx_hbm.at[pl.ds(base_w + g * G, G)], idx_v.at[b])
      for j in range(G):
        pltpu.async_copy(
            table_hbm.at[idx_v.at[b].at[j]], rows_v.at[b].at[j], sems[b])

    def drain(b):
      for j in range(G):
        pltpu.make_async_copy(
            table_hbm.at[idx_v.at[b].at[j]], rows_v.at[b].at[j],
            sems[b]).wait()

    def store(b, g):
      pltpu.sync_copy(rows_v.at[b], out_hbm.at[pl.ds(base_w + g * G, G)])

    load_fire(0, 0)

    def outer(o, _):
      g0 = 2 * o
      load_fire(1, g0 + 1)
      drain(0)
      store(0, g0)

      @pl.when(g0 + 2 < groups)
      def _():
        load_fire(0, g0 + 2)

      drain(1)
      store(1, g0 + 1)
      return ()

    lax.fori_loop(0, groups // 2, outer, ())

  return k


def kernel(x, table):
  B = x.size
  idx = x.reshape(B // ROW, ROW).astype(jnp.int32)
  tbl = _transpose_table(table.T)
  out = _make_gather(B)(idx, tbl)
  return out[:, :, :D_MODEL].reshape(x.shape + (D_MODEL,))


# speedup vs baseline: 1.0456x; 1.0456x over previous
import functools

import jax
import jax.numpy as jnp
from jax import lax
from jax.experimental import pallas as pl
from jax.experimental.pallas import tpu as pltpu
from jax.experimental.pallas import tpu_sc as plsc

D_MODEL = 64
NC, NS = 2, 16
NW = NC * NS
ROW = 128
G = 5
TCOLS = 2048


def _transpose_block(in_ref, out_ref):
  x = in_ref[...]
  rows = lax.broadcasted_iota(jnp.int32, (D_MODEL, 2 * D_MODEL), 0)
  cols = lax.broadcasted_iota(jnp.int32, (D_MODEL, 2 * D_MODEL), 1)
  w = jnp.where(cols % D_MODEL == rows, 1.0, 0.0).astype(jnp.float32)
  out_ref[...] = lax.dot_general(
      x, w, (((0,), (0,)), ((), ())),
      preferred_element_type=jnp.float32,
      precision=lax.Precision.HIGHEST)


def _transpose_table(table_t):
  V = table_t.shape[1]
  steps = (V + TCOLS - 1) // TCOLS
  return pl.pallas_call(
      _transpose_block,
      grid=(steps,),
      in_specs=[pl.BlockSpec((D_MODEL, TCOLS), lambda i: (0, i))],
      out_specs=pl.BlockSpec((TCOLS, 2 * D_MODEL), lambda i: (i, 0)),
      out_shape=jax.ShapeDtypeStruct((V, 2 * D_MODEL), jnp.float32),
  )(table_t)


def _make_gather(B: int):
  rows_total = B // ROW
  rows_per_w = rows_total // NW
  groups = rows_per_w // G

  mesh = plsc.VectorSubcoreMesh(core_axis_name="c", subcore_axis_name="s")

  @functools.partial(
      pl.kernel,
      mesh=mesh,
      compiler_params=pltpu.CompilerParams(use_tc_tiling_on_sc=False),
      out_type=jax.ShapeDtypeStruct((rows_total, ROW, 2 * D_MODEL),
                                    jnp.float32),
      scratch_types=[
          pltpu.VMEM((2, G, ROW), jnp.int32),
          pltpu.VMEM((2, G, ROW, D_MODEL), jnp.float32),
          pltpu.SemaphoreType.DMA,
          pltpu.SemaphoreType.DMA,
      ],
  )
  def k(idx_hbm, table_hbm, out_hbm, idx_v, rows_v, gsem0, gsem1):
    wid = lax.axis_index("s") * NC + lax.axis_index("c")
    base_w = wid * rows_per_w
    sems = (gsem0, gsem1)

    def load_fire(b, g):
      pltpu.sync_copy(idx_hbm.at[pl.ds(base_w + g * G, G)], idx_v.at[b])
      for j in range(G):
        pltpu.async_copy(
            table_hbm.at[idx_v.at[b].at[j]], rows_v.at[b].at[j], sems[b])

    def drain(b):
      for j in range(G):
        pltpu.make_async_copy(
            table_hbm.at[idx_v.at[b].at[j]], rows_v.at[b].at[j],
            sems[b]).wait()

    def store(b, g):
      pltpu.sync_copy(
          rows_v.at[b],
          out_hbm.at[pl.ds(base_w + g * G, G), :, pl.ds(0, D_MODEL)])

    load_fire(0, 0)

    def outer(o, _):
      g0 = 2 * o
      load_fire(1, g0 + 1)
      drain(0)
      store(0, g0)

      @pl.when(g0 + 2 < groups)
      def _():
        load_fire(0, g0 + 2)

      drain(1)
      store(1, g0 + 1)
      return ()

    lax.fori_loop(0, groups // 2, outer, ())

  return k


def kernel(x, table):
  B = x.size
  idx = x.reshape(B // ROW, ROW).astype(jnp.int32) * 2
  tbl = _transpose_table(table.T).reshape(2 * table.shape[0], D_MODEL)
  out = _make_gather(B)(idx, tbl)
  return out[:, :, :D_MODEL].reshape(x.shape + (D_MODEL,))


# speedup vs baseline: 1.2379x; 1.1839x over previous
import functools

import jax
import jax.numpy as jnp
from jax import lax
from jax.experimental import pallas as pl
from jax.experimental.pallas import tpu as pltpu
from jax.experimental.pallas import tpu_sc as plsc

D_MODEL = 64
NC, NS = 2, 16
NW = NC * NS
ROW = 128
G = 5
TCOLS = 2048


def _transpose_block(in_ref, out_ref):
  x = in_ref[...]
  out_ref[:, 0:D_MODEL] = x.T


def _transpose_table(table_t):
  V = table_t.shape[1]
  steps = (V + TCOLS - 1) // TCOLS
  return pl.pallas_call(
      _transpose_block,
      grid=(steps,),
      in_specs=[pl.BlockSpec((D_MODEL, TCOLS), lambda i: (0, i))],
      out_specs=pl.BlockSpec((TCOLS, 2 * D_MODEL), lambda i: (i, 0)),
      out_shape=jax.ShapeDtypeStruct((V, 2 * D_MODEL), jnp.float32),
  )(table_t)


def _make_gather(B: int):
  rows_total = B // ROW
  rows_per_w = rows_total // NW
  groups = rows_per_w // G

  mesh = plsc.VectorSubcoreMesh(core_axis_name="c", subcore_axis_name="s")

  @functools.partial(
      pl.kernel,
      mesh=mesh,
      compiler_params=pltpu.CompilerParams(use_tc_tiling_on_sc=False),
      out_type=jax.ShapeDtypeStruct((rows_total, ROW, 2 * D_MODEL),
                                    jnp.float32),
      scratch_types=[
          pltpu.VMEM((2, G, ROW), jnp.int32),
          pltpu.VMEM((2, G, ROW, D_MODEL), jnp.float32),
          pltpu.SemaphoreType.DMA,
          pltpu.SemaphoreType.DMA,
      ],
  )
  def k(idx_hbm, table_hbm, out_hbm, idx_v, rows_v, gsem0, gsem1):
    wid = lax.axis_index("s") * NC + lax.axis_index("c")
    base_w = wid * rows_per_w
    sems = (gsem0, gsem1)

    def load_fire(b, g):
      pltpu.sync_copy(idx_hbm.at[pl.ds(base_w + g * G, G)], idx_v.at[b])
      for j in range(G):
        pltpu.async_copy(
            table_hbm.at[idx_v.at[b].at[j]], rows_v.at[b].at[j], sems[b])

    def drain(b):
      for j in range(G):
        pltpu.make_async_copy(
            table_hbm.at[idx_v.at[b].at[j]], rows_v.at[b].at[j],
            sems[b]).wait()

    def store(b, g):
      pltpu.sync_copy(
          rows_v.at[b],
          out_hbm.at[pl.ds(base_w + g * G, G), :, pl.ds(0, D_MODEL)])

    load_fire(0, 0)

    def outer(o, _):
      g0 = 2 * o
      load_fire(1, g0 + 1)
      drain(0)
      store(0, g0)

      @pl.when(g0 + 2 < groups)
      def _():
        load_fire(0, g0 + 2)

      drain(1)
      store(1, g0 + 1)
      return ()

    lax.fori_loop(0, groups // 2, outer, ())

  return k


def kernel(x, table):
  B = x.size
  idx = x.reshape(B // ROW, ROW).astype(jnp.int32) * 2
  tbl = _transpose_table(table.T).reshape(2 * table.shape[0], D_MODEL)
  out = _make_gather(B)(idx, tbl)
  return out[:, :, :D_MODEL].reshape(x.shape + (D_MODEL,))


# speedup vs baseline: 1.6392x; 1.3241x over previous
import functools

import jax
import jax.numpy as jnp
from jax import lax
from jax.experimental import pallas as pl
from jax.experimental.pallas import tpu as pltpu
from jax.experimental.pallas import tpu_sc as plsc

D_MODEL = 64
NC, NS = 2, 16
NW = NC * NS
ROW = 128
G = 5
TCOLS = 8192


def _transpose_block(in_ref, out_ref):
  x = in_ref[...]
  out_ref[:, 0:D_MODEL] = x.T


def _transpose_table(table_t):
  V = table_t.shape[1]
  steps = (V + TCOLS - 1) // TCOLS
  return pl.pallas_call(
      _transpose_block,
      grid=(steps,),
      in_specs=[pl.BlockSpec((D_MODEL, TCOLS), lambda i: (0, i))],
      out_specs=pl.BlockSpec((TCOLS, 2 * D_MODEL), lambda i: (i, 0)),
      out_shape=jax.ShapeDtypeStruct((V, 2 * D_MODEL), jnp.float32),
  )(table_t)


def _make_gather(B: int):
  rows_total = B // ROW
  rows_per_w = rows_total // NW
  groups = rows_per_w // G

  mesh = plsc.VectorSubcoreMesh(core_axis_name="c", subcore_axis_name="s")

  @functools.partial(
      pl.kernel,
      mesh=mesh,
      compiler_params=pltpu.CompilerParams(use_tc_tiling_on_sc=False),
      out_type=jax.ShapeDtypeStruct((rows_total, ROW, 2 * D_MODEL),
                                    jnp.float32),
      scratch_types=[
          pltpu.VMEM((2, G, ROW), jnp.int32),
          pltpu.VMEM((2, G, ROW, D_MODEL), jnp.float32),
          pltpu.SemaphoreType.DMA,
          pltpu.SemaphoreType.DMA,
      ],
  )
  def k(idx_hbm, table_hbm, out_hbm, idx_v, rows_v, gsem0, gsem1):
    wid = lax.axis_index("s") * NC + lax.axis_index("c")
    base_w = wid * rows_per_w
    sems = (gsem0, gsem1)

    def load_fire(b, g):
      pltpu.sync_copy(idx_hbm.at[pl.ds(base_w + g * G, G)], idx_v.at[b])
      for j in range(G):
        pltpu.async_copy(
            table_hbm.at[idx_v.at[b].at[j]], rows_v.at[b].at[j], sems[b])

    def drain(b):
      for j in range(G):
        pltpu.make_async_copy(
            table_hbm.at[idx_v.at[b].at[j]], rows_v.at[b].at[j],
            sems[b]).wait()

    def store(b, g):
      pltpu.sync_copy(
          rows_v.at[b],
          out_hbm.at[pl.ds(base_w + g * G, G), :, pl.ds(0, D_MODEL)])

    load_fire(0, 0)

    def outer(o, _):
      g0 = 2 * o
      load_fire(1, g0 + 1)
      drain(0)
      store(0, g0)

      @pl.when(g0 + 2 < groups)
      def _():
        load_fire(0, g0 + 2)

      drain(1)
      store(1, g0 + 1)
      return ()

    lax.fori_loop(0, groups // 2, outer, ())

  return k


def kernel(x, table):
  B = x.size
  idx = x.reshape(B // ROW, ROW).astype(jnp.int32) * 2
  tbl = _transpose_table(table.T).reshape(2 * table.shape[0], D_MODEL)
  out = _make_gather(B)(idx, tbl)
  return out[:, :, :D_MODEL].reshape(x.shape + (D_MODEL,))


# speedup vs baseline: 1.6855x; 1.0283x over previous
import functools

import jax
import jax.numpy as jnp
from jax import lax
from jax.experimental import pallas as pl
from jax.experimental.pallas import tpu as pltpu
from jax.experimental.pallas import tpu_sc as plsc

D_MODEL = 64
NC, NS = 2, 16
NW = NC * NS
ROW = 128
G = 5
TCOLS = 16384


def _transpose_block(in_ref, out_ref):
  x = in_ref[...]
  out_ref[:, 0:D_MODEL] = x.T


def _transpose_table(table_t):
  V = table_t.shape[1]
  steps = (V + TCOLS - 1) // TCOLS
  return pl.pallas_call(
      _transpose_block,
      grid=(steps,),
      in_specs=[pl.BlockSpec((D_MODEL, TCOLS), lambda i: (0, i))],
      out_specs=pl.BlockSpec((TCOLS, 2 * D_MODEL), lambda i: (i, 0)),
      out_shape=jax.ShapeDtypeStruct((V, 2 * D_MODEL), jnp.float32),
  )(table_t)


def _make_gather(B: int):
  rows_total = B // ROW
  rows_per_w = rows_total // NW
  groups = rows_per_w // G

  mesh = plsc.VectorSubcoreMesh(core_axis_name="c", subcore_axis_name="s")

  @functools.partial(
      pl.kernel,
      mesh=mesh,
      compiler_params=pltpu.CompilerParams(use_tc_tiling_on_sc=False),
      out_type=jax.ShapeDtypeStruct((rows_total, ROW, 2 * D_MODEL),
                                    jnp.float32),
      scratch_types=[
          pltpu.VMEM((2, G, ROW), jnp.int32),
          pltpu.VMEM((2, G, ROW, D_MODEL), jnp.float32),
          pltpu.SemaphoreType.DMA,
          pltpu.SemaphoreType.DMA,
      ],
  )
  def k(idx_hbm, table_hbm, out_hbm, idx_v, rows_v, gsem0, gsem1):
    wid = lax.axis_index("s") * NC + lax.axis_index("c")
    base_w = wid * rows_per_w
    sems = (gsem0, gsem1)

    def load_fire(b, g):
      pltpu.sync_copy(idx_hbm.at[pl.ds(base_w + g * G, G)], idx_v.at[b])
      for j in range(G):
        pltpu.async_copy(
            table_hbm.at[idx_v.at[b].at[j]], rows_v.at[b].at[j], sems[b])

    def drain(b):
      for j in range(G):
        pltpu.make_async_copy(
            table_hbm.at[idx_v.at[b].at[j]], rows_v.at[b].at[j],
            sems[b]).wait()

    def store(b, g):
      pltpu.sync_copy(
          rows_v.at[b],
          out_hbm.at[pl.ds(base_w + g * G, G), :, pl.ds(0, D_MODEL)])

    load_fire(0, 0)

    def outer(o, _):
      g0 = 2 * o
      load_fire(1, g0 + 1)
      drain(0)
      store(0, g0)

      @pl.when(g0 + 2 < groups)
      def _():
        load_fire(0, g0 + 2)

      drain(1)
      store(1, g0 + 1)
      return ()

    lax.fori_loop(0, groups // 2, outer, ())

  return k


def kernel(x, table):
  B = x.size
  idx = x.reshape(B // ROW, ROW).astype(jnp.int32) * 2
  tbl = _transpose_table(table.T).reshape(2 * table.shape[0], D_MODEL)
  out = _make_gather(B)(idx, tbl)
  return out[:, :, :D_MODEL].reshape(x.shape + (D_MODEL,))


# speedup vs baseline: 1.7027x; 1.0102x over previous
import functools

import jax
import jax.numpy as jnp
from jax import lax
from jax.experimental import pallas as pl
from jax.experimental.pallas import tpu as pltpu
from jax.experimental.pallas import tpu_sc as plsc

D_MODEL = 64
NC, NS = 2, 16
NW = NC * NS
ROW = 128
G = 5
TCOLS = 32768


def _transpose_block(in_ref, out_ref):
  x = in_ref[...]
  out_ref[:, 0:D_MODEL] = x.T


def _transpose_table(table_t):
  V = table_t.shape[1]
  steps = (V + TCOLS - 1) // TCOLS
  return pl.pallas_call(
      _transpose_block,
      grid=(steps,),
      in_specs=[pl.BlockSpec((D_MODEL, TCOLS), lambda i: (0, i))],
      out_specs=pl.BlockSpec((TCOLS, 2 * D_MODEL), lambda i: (i, 0)),
      out_shape=jax.ShapeDtypeStruct((V, 2 * D_MODEL), jnp.float32),
  )(table_t)


def _make_gather(B: int):
  rows_total = B // ROW
  rows_per_w = rows_total // NW
  groups = rows_per_w // G

  mesh = plsc.VectorSubcoreMesh(core_axis_name="c", subcore_axis_name="s")

  @functools.partial(
      pl.kernel,
      mesh=mesh,
      compiler_params=pltpu.CompilerParams(use_tc_tiling_on_sc=False),
      out_type=jax.ShapeDtypeStruct((rows_total, ROW, 2 * D_MODEL),
                                    jnp.float32),
      scratch_types=[
          pltpu.VMEM((2, G, ROW), jnp.int32),
          pltpu.VMEM((2, G, ROW, D_MODEL), jnp.float32),
          pltpu.SemaphoreType.DMA,
          pltpu.SemaphoreType.DMA,
      ],
  )
  def k(idx_hbm, table_hbm, out_hbm, idx_v, rows_v, gsem0, gsem1):
    wid = lax.axis_index("s") * NC + lax.axis_index("c")
    base_w = wid * rows_per_w
    sems = (gsem0, gsem1)

    def load_fire(b, g):
      pltpu.sync_copy(idx_hbm.at[pl.ds(base_w + g * G, G)], idx_v.at[b])
      for j in range(G):
        pltpu.async_copy(
            table_hbm.at[idx_v.at[b].at[j]], rows_v.at[b].at[j], sems[b])

    def drain(b):
      for j in range(G):
        pltpu.make_async_copy(
            table_hbm.at[idx_v.at[b].at[j]], rows_v.at[b].at[j],
            sems[b]).wait()

    def store(b, g):
      pltpu.sync_copy(
          rows_v.at[b],
          out_hbm.at[pl.ds(base_w + g * G, G), :, pl.ds(0, D_MODEL)])

    load_fire(0, 0)

    def outer(o, _):
      g0 = 2 * o
      load_fire(1, g0 + 1)
      drain(0)
      store(0, g0)

      @pl.when(g0 + 2 < groups)
      def _():
        load_fire(0, g0 + 2)

      drain(1)
      store(1, g0 + 1)
      return ()

    lax.fori_loop(0, groups // 2, outer, ())

  return k


def kernel(x, table):
  B = x.size
  idx = x.reshape(B // ROW, ROW).astype(jnp.int32) * 2
  tbl = _transpose_table(table.T).reshape(2 * table.shape[0], D_MODEL)
  out = _make_gather(B)(idx, tbl)
  return out[:, :, :D_MODEL].reshape(x.shape + (D_MODEL,))


# speedup vs baseline: 1.7240x; 1.0125x over previous
import functools

import jax
import jax.numpy as jnp
from jax import lax
from jax.experimental import pallas as pl
from jax.experimental.pallas import tpu as pltpu
from jax.experimental.pallas import tpu_sc as plsc

D_MODEL = 64
NC, NS = 2, 16
NW = NC * NS
ROW = 128
G = 5
TCOLS = 8192


SPLIT = 64 * TCOLS


def _transpose_block(a_ref, b_ref, out_ref):
  out_ref[:, 0:D_MODEL] = a_ref[...].T
  out_ref[:, D_MODEL:2 * D_MODEL] = b_ref[...].T


def _transpose_table(table_t):
  nblk = SPLIT // TCOLS
  last_blk = (table_t.shape[1] - 1) // TCOLS
  return pl.pallas_call(
      _transpose_block,
      grid=(nblk,),
      in_specs=[
          pl.BlockSpec((D_MODEL, TCOLS), lambda i: (0, i)),
          pl.BlockSpec((D_MODEL, TCOLS),
                       lambda i: (0, jnp.minimum(i + nblk, last_blk))),
      ],
      out_specs=pl.BlockSpec((TCOLS, 2 * D_MODEL), lambda i: (i, 0)),
      out_shape=jax.ShapeDtypeStruct((SPLIT, 2 * D_MODEL), jnp.float32),
  )(table_t, table_t)


def _make_gather(B: int):
  rows_total = B // ROW
  rows_per_w = rows_total // NW
  groups = rows_per_w // G

  mesh = plsc.VectorSubcoreMesh(core_axis_name="c", subcore_axis_name="s")

  @functools.partial(
      pl.kernel,
      mesh=mesh,
      compiler_params=pltpu.CompilerParams(use_tc_tiling_on_sc=False),
      out_type=jax.ShapeDtypeStruct((rows_total, ROW, 2 * D_MODEL),
                                    jnp.float32),
      scratch_types=[
          pltpu.VMEM((2, G, ROW), jnp.int32),
          pltpu.VMEM((2, G, ROW, D_MODEL), jnp.float32),
          pltpu.SemaphoreType.DMA,
          pltpu.SemaphoreType.DMA,
      ],
  )
  def k(idx_hbm, table_hbm, out_hbm, idx_v, rows_v, gsem0, gsem1):
    wid = lax.axis_index("s") * NC + lax.axis_index("c")
    base_w = wid * rows_per_w
    sems = (gsem0, gsem1)

    def load_fire(b, g):
      pltpu.sync_copy(idx_hbm.at[pl.ds(base_w + g * G, G)], idx_v.at[b])
      for j in range(G):
        pltpu.async_copy(
            table_hbm.at[idx_v.at[b].at[j]], rows_v.at[b].at[j], sems[b])

    def drain(b):
      for j in range(G):
        pltpu.make_async_copy(
            table_hbm.at[idx_v.at[b].at[j]], rows_v.at[b].at[j],
            sems[b]).wait()

    def store(b, g):
      pltpu.sync_copy(
          rows_v.at[b],
          out_hbm.at[pl.ds(base_w + g * G, G), :, pl.ds(0, D_MODEL)])

    load_fire(0, 0)

    def outer(o, _):
      g0 = 2 * o
      load_fire(1, g0 + 1)
      drain(0)
      store(0, g0)

      @pl.when(g0 + 2 < groups)
      def _():
        load_fire(0, g0 + 2)

      drain(1)
      store(1, g0 + 1)
      return ()

    lax.fori_loop(0, groups // 2, outer, ())

  return k


def kernel(x, table):
  B = x.size
  xi = x.reshape(B // ROW, ROW).astype(jnp.int32)
  idx = 2 * xi - jnp.where(xi >= SPLIT, 2 * SPLIT - 1, 0)
  tbl = _transpose_table(table.T).reshape(2 * SPLIT, D_MODEL)
  out = _make_gather(B)(idx, tbl)
  return out[:, :, :D_MODEL].reshape(x.shape + (D_MODEL,))


# speedup vs baseline: 1.7689x; 1.0261x over previous
import functools

import jax
import jax.numpy as jnp
from jax import lax
from jax.experimental import pallas as pl
from jax.experimental.pallas import tpu as pltpu
from jax.experimental.pallas import tpu_sc as plsc

D_MODEL = 64
NC, NS = 2, 16
NW = NC * NS
ROW = 128
G = 5
TCOLS = 16384


SPLIT = 32 * TCOLS


def _transpose_block(a_ref, b_ref, out_ref):
  out_ref[:, 0:D_MODEL] = a_ref[...].T
  out_ref[:, D_MODEL:2 * D_MODEL] = b_ref[...].T


def _transpose_table(table_t):
  nblk = SPLIT // TCOLS
  last_blk = (table_t.shape[1] - 1) // TCOLS
  return pl.pallas_call(
      _transpose_block,
      grid=(nblk,),
      in_specs=[
          pl.BlockSpec((D_MODEL, TCOLS), lambda i: (0, i)),
          pl.BlockSpec((D_MODEL, TCOLS),
                       lambda i: (0, jnp.minimum(i + nblk, last_blk))),
      ],
      out_specs=pl.BlockSpec((TCOLS, 2 * D_MODEL), lambda i: (i, 0)),
      out_shape=jax.ShapeDtypeStruct((SPLIT, 2 * D_MODEL), jnp.float32),
  )(table_t, table_t)


def _make_gather(B: int):
  rows_total = B // ROW
  rows_per_w = rows_total // NW
  groups = rows_per_w // G

  mesh = plsc.VectorSubcoreMesh(core_axis_name="c", subcore_axis_name="s")

  @functools.partial(
      pl.kernel,
      mesh=mesh,
      compiler_params=pltpu.CompilerParams(use_tc_tiling_on_sc=False),
      out_type=jax.ShapeDtypeStruct((rows_total, ROW, 2 * D_MODEL),
                                    jnp.float32),
      scratch_types=[
          pltpu.VMEM((2, G, ROW), jnp.int32),
          pltpu.VMEM((2, G, ROW, D_MODEL), jnp.float32),
          pltpu.SemaphoreType.DMA,
          pltpu.SemaphoreType.DMA,
      ],
  )
  def k(idx_hbm, table_hbm, out_hbm, idx_v, rows_v, gsem0, gsem1):
    wid = lax.axis_index("s") * NC + lax.axis_index("c")
    base_w = wid * rows_per_w
    sems = (gsem0, gsem1)

    def load_fire(b, g):
      pltpu.sync_copy(idx_hbm.at[pl.ds(base_w + g * G, G)], idx_v.at[b])
      for j in range(G):
        pltpu.async_copy(
            table_hbm.at[idx_v.at[b].at[j]], rows_v.at[b].at[j], sems[b])

    def drain(b):
      for j in range(G):
        pltpu.make_async_copy(
            table_hbm.at[idx_v.at[b].at[j]], rows_v.at[b].at[j],
            sems[b]).wait()

    def store(b, g):
      pltpu.sync_copy(
          rows_v.at[b],
          out_hbm.at[pl.ds(base_w + g * G, G), :, pl.ds(0, D_MODEL)])

    load_fire(0, 0)

    def outer(o, _):
      g0 = 2 * o
      load_fire(1, g0 + 1)
      drain(0)
      store(0, g0)

      @pl.when(g0 + 2 < groups)
      def _():
        load_fire(0, g0 + 2)

      drain(1)
      store(1, g0 + 1)
      return ()

    lax.fori_loop(0, groups // 2, outer, ())

  return k


def kernel(x, table):
  B = x.size
  xi = x.reshape(B // ROW, ROW).astype(jnp.int32)
  idx = 2 * xi - jnp.where(xi >= SPLIT, 2 * SPLIT - 1, 0)
  tbl = _transpose_table(table.T).reshape(2 * SPLIT, D_MODEL)
  out = _make_gather(B)(idx, tbl)
  return out[:, :, :D_MODEL].reshape(x.shape + (D_MODEL,))
